# Initial kernel scaffold; baseline (speedup 1.0000x reference)
import jax
import jax.numpy as jnp
from jax.experimental import pallas as pl


def kernel(xyz, points, params):
    raise NotImplementedError("write your pallas kernel here")


# speedup vs baseline: 2.8762x; 2.8762x over previous
import functools

import jax
import jax.numpy as jnp
from jax.experimental import pallas as pl
from jax.experimental.pallas import tpu as pltpu

_NPOINT = 1024
_RADII = (0.1, 0.2, 0.4)
_NSAMPLES = (16, 32, 64)
_NS_TOT = 16 + 32 + 64
_EPS = 1e-3


def _fps_body(xs_ref, ys_ref, zs_ref, ox_ref, oy_ref, oz_ref, dists_ref):
    b, n = xs_ref.shape
    xs = xs_ref[...]
    ys = ys_ref[...]
    zs = zs_ref[...]
    lane = jax.lax.broadcasted_iota(jnp.int32, (1, n), 1)
    lane_o = jax.lax.broadcasted_iota(jnp.int32, (1, _NPOINT), 1)
    dists_ref[...] = jnp.full((b, n), 1e10, jnp.float32)

    def step(i, far):
        onehot = (lane == far).astype(jnp.float32)
        cx = jnp.sum(xs * onehot, axis=1, keepdims=True)
        cy = jnp.sum(ys * onehot, axis=1, keepdims=True)
        cz = jnp.sum(zs * onehot, axis=1, keepdims=True)
        d = (xs - cx) ** 2 + (ys - cy) ** 2 + (zs - cz) ** 2
        dmin = jnp.minimum(dists_ref[...], d)
        dists_ref[...] = dmin
        m = jnp.max(dmin, axis=1, keepdims=True)
        nxt = jnp.min(jnp.where(dmin == m, lane, n), axis=1, keepdims=True)
        sel = lane_o == i
        ox_ref[...] = jnp.where(sel, cx, ox_ref[...])
        oy_ref[...] = jnp.where(sel, cy, oy_ref[...])
        oz_ref[...] = jnp.where(sel, cz, oz_ref[...])
        return nxt

    far0 = jnp.zeros((b, 1), jnp.int32)
    jax.lax.fori_loop(0, _NPOINT, step, far0)


def _run_fps(xyz):
    b, n, _ = xyz.shape
    xs = xyz[:, :, 0]
    ys = xyz[:, :, 1]
    zs = xyz[:, :, 2]
    out = pl.pallas_call(
        _fps_body,
        out_shape=[jax.ShapeDtypeStruct((b, _NPOINT), jnp.float32)] * 3,
        scratch_shapes=[pltpu.VMEM((b, n), jnp.float32)],
    )(xs, ys, zs)
    return out


def _cumsum_lanes(v):
    n = v.shape[1]
    sh = 1
    while sh < n:
        v = v + jnp.pad(v[:, : n - sh], ((0, 0), (sh, 0)))
        sh *= 2
    return v


def _group_body(xs_ref, ys_ref, zs_ref, qx_ref, qy_ref, qz_ref, feat_ref,
                out_ref, *, s_blk):
    n = xs_ref.shape[2]
    xs = xs_ref[0]
    ys = ys_ref[0]
    zs = zs_ref[0]
    lane = jax.lax.broadcasted_iota(jnp.int32, (1, n), 1)
    e0 = (lane == 0).astype(jnp.float32)
    feat = feat_ref[0]

    for k in range(s_blk):
        qx = qx_ref[0, k, 0]
        qy = qy_ref[0, k, 0]
        qz = qz_ref[0, k, 0]
        d = (xs - qx) ** 2 + (ys - qy) ** 2 + (zs - qz) ** 2
        rows = []
        for r, ns in zip(_RADII, _NSAMPLES):
            mask = (d <= r * r).astype(jnp.float32)
            c = _cumsum_lanes(mask)
            cnt = c[0, n - 1]
            kv = jax.lax.broadcasted_iota(
                jnp.int32, (ns, 1), 0).astype(jnp.float32)
            a = ((c == (kv + 1.0)) & (mask > 0.0)).astype(jnp.float32)
            row0 = ((c == 1.0) & (mask > 0.0)).astype(jnp.float32)
            has = (cnt > 0.0).astype(jnp.float32)
            row0 = has * row0 + (1.0 - has) * e0
            a = a + (kv >= cnt).astype(jnp.float32) * row0
            rows.append(a)
        amat = jnp.concatenate(rows, axis=0)
        g = jax.lax.dot_general(
            amat, feat, (((1,), (0,)), ((), ())),
            precision=jax.lax.Precision.HIGHEST,
            preferred_element_type=jnp.float32)
        q = jnp.concatenate(
            [jnp.full((1, 1), qx, jnp.float32),
             jnp.full((1, 1), qy, jnp.float32),
             jnp.full((1, 1), qz, jnp.float32)], axis=1)
        g = g - jnp.pad(q, ((0, 0), (32, 0)))
        out_ref[0, k] = g


def _run_group(xs, ys, zs, qx, qy, qz, feat, s_blk=8):
    b, n = xs.shape
    grid = (b, _NPOINT // s_blk)
    kern = functools.partial(_group_body, s_blk=s_blk)
    out = pl.pallas_call(
        kern,
        grid=grid,
        in_specs=[
            pl.BlockSpec((1, 1, n), lambda i, j: (i, 0, 0)),
            pl.BlockSpec((1, 1, n), lambda i, j: (i, 0, 0)),
            pl.BlockSpec((1, 1, n), lambda i, j: (i, 0, 0)),
            pl.BlockSpec((1, s_blk, 1), lambda i, j: (i, j, 0)),
            pl.BlockSpec((1, s_blk, 1), lambda i, j: (i, j, 0)),
            pl.BlockSpec((1, s_blk, 1), lambda i, j: (i, j, 0)),
            pl.BlockSpec((1, n, 35), lambda i, j: (i, 0, 0)),
        ],
        out_specs=pl.BlockSpec((1, s_blk, _NS_TOT, 35),
                               lambda i, j: (i, j, 0, 0)),
        out_shape=jax.ShapeDtypeStruct((b, _NPOINT, _NS_TOT, 35),
                                       jnp.float32),
    )(xs[:, None, :], ys[:, None, :], zs[:, None, :],
      qx[:, :, None], qy[:, :, None], qz[:, :, None], feat)
    return out


def _conv_stats_body(x_ref, w_ref, y_ref, st_ref):
    i = pl.program_id(0)
    y = jax.lax.dot_general(
        x_ref[...], w_ref[...], (((1,), (0,)), ((), ())),
        preferred_element_type=jnp.float32)
    y_ref[...] = y
    s = jnp.sum(y, axis=0, keepdims=True)
    s2 = jnp.sum(y * y, axis=0, keepdims=True)
    st = jnp.concatenate([s, s2], axis=0)

    @pl.when(i == 0)
    def _():
        st_ref[...] = jnp.zeros_like(st_ref)

    st_ref[...] += st


def _norm_conv_stats_body(x_ref, st_in_ref, g_ref, b_ref, w_ref,
                          y_ref, st_ref, *, m_total):
    i = pl.program_id(0)
    st_in = st_in_ref[...]
    mean = st_in[0:1] / m_total
    var = st_in[1:2] / m_total - mean * mean
    scale = g_ref[...] * jax.lax.rsqrt(var + _EPS)
    bias = b_ref[...] - mean * scale
    z = jnp.maximum(x_ref[...] * scale + bias, 0.0)
    y = jax.lax.dot_general(
        z, w_ref[...], (((1,), (0,)), ((), ())),
        preferred_element_type=jnp.float32)
    y_ref[...] = y
    s = jnp.sum(y, axis=0, keepdims=True)
    s2 = jnp.sum(y * y, axis=0, keepdims=True)
    st = jnp.concatenate([s, s2], axis=0)

    @pl.when(i == 0)
    def _():
        st_ref[...] = jnp.zeros_like(st_ref)

    st_ref[...] += st


def _norm_pool_body(x_ref, st_in_ref, g_ref, b_ref, o_ref, *, m_total, ns):
    st_in = st_in_ref[...]
    mean = st_in[0:1] / m_total
    var = st_in[1:2] / m_total - mean * mean
    scale = g_ref[...] * jax.lax.rsqrt(var + _EPS)
    bias = b_ref[...] - mean * scale
    z = jnp.maximum(x_ref[...] * scale + bias, 0.0)
    rows, c = z.shape
    o_ref[...] = jnp.max(z.reshape(rows // ns, ns, c), axis=1)


def _conv_stats(x, w, blk):
    m, cin = x.shape
    cout = w.shape[1]
    grid = (m // blk,)
    return pl.pallas_call(
        _conv_stats_body,
        grid=grid,
        in_specs=[
            pl.BlockSpec((blk, cin), lambda i: (i, 0)),
            pl.BlockSpec((cin, cout), lambda i: (0, 0)),
        ],
        out_specs=[
            pl.BlockSpec((blk, cout), lambda i: (i, 0)),
            pl.BlockSpec((2, cout), lambda i: (0, 0)),
        ],
        out_shape=[
            jax.ShapeDtypeStruct((m, cout), jnp.float32),
            jax.ShapeDtypeStruct((2, cout), jnp.float32),
        ],
    )(x, w)


def _norm_conv_stats(x, st, gamma, beta, w, blk):
    m, cin = x.shape
    cout = w.shape[1]
    grid = (m // blk,)
    kern = functools.partial(_norm_conv_stats_body, m_total=float(m))
    return pl.pallas_call(
        kern,
        grid=grid,
        in_specs=[
            pl.BlockSpec((blk, cin), lambda i: (i, 0)),
            pl.BlockSpec((2, cin), lambda i: (0, 0)),
            pl.BlockSpec((1, cin), lambda i: (0, 0)),
            pl.BlockSpec((1, cin), lambda i: (0, 0)),
            pl.BlockSpec((cin, cout), lambda i: (0, 0)),
        ],
        out_specs=[
            pl.BlockSpec((blk, cout), lambda i: (i, 0)),
            pl.BlockSpec((2, cout), lambda i: (0, 0)),
        ],
        out_shape=[
            jax.ShapeDtypeStruct((m, cout), jnp.float32),
            jax.ShapeDtypeStruct((2, cout), jnp.float32),
        ],
    )(x, st, gamma, beta, w)


def _norm_pool(x, st, gamma, beta, ns, blk):
    m, c = x.shape
    grid = (m // blk,)
    kern = functools.partial(_norm_pool_body, m_total=float(m), ns=ns)
    return pl.pallas_call(
        kern,
        grid=grid,
        in_specs=[
            pl.BlockSpec((blk, c), lambda i: (i, 0)),
            pl.BlockSpec((2, c), lambda i: (0, 0)),
            pl.BlockSpec((1, c), lambda i: (0, 0)),
            pl.BlockSpec((1, c), lambda i: (0, 0)),
        ],
        out_specs=pl.BlockSpec((blk // ns, c), lambda i: (i, 0)),
        out_shape=jax.ShapeDtypeStruct((m // ns, c), jnp.float32),
    )(x, st, gamma, beta)


def kernel(xyz, points, params):
    b, n, _ = xyz.shape
    ox, oy, oz = _run_fps(xyz)

    feat = jnp.concatenate([points, xyz], axis=-1)
    grouped = _run_group(xyz[:, :, 0], xyz[:, :, 1], xyz[:, :, 2],
                         ox, oy, oz, feat)

    outs = []
    off = 0
    for si, ns in enumerate(_NSAMPLES):
        x = grouped[:, :, off:off + ns, :].reshape(b * _NPOINT * ns, 35)
        off += ns
        blk = min(4096, x.shape[0])
        p0, p1, p2 = params[si]
        y, st = _conv_stats(x, p0["W"], blk)
        y, st2 = _norm_conv_stats(y, st, p0["gamma"][None], p0["beta"][None],
                                  p1["W"], blk)
        y, st3 = _norm_conv_stats(y, st2, p1["gamma"][None], p1["beta"][None],
                                  p2["W"], blk)
        o = _norm_pool(y, st3, p2["gamma"][None], p2["beta"][None], ns, blk)
        outs.append(o.reshape(b, _NPOINT, -1))

    new_xyz = jnp.stack([ox, oy, oz], axis=-1)
    return new_xyz, jnp.concatenate(outs, axis=-1)


# speedup vs baseline: 5.2858x; 1.8378x over previous
import functools

import jax
import jax.numpy as jnp
from jax.experimental import pallas as pl
from jax.experimental.pallas import tpu as pltpu

_NPOINT = 1024
_RADII = (0.1, 0.2, 0.4)
_NSAMPLES = (16, 32, 64)
_NS_TOT = 16 + 32 + 64
_EPS = 1e-3


def _fps_body(xs_ref, ys_ref, zs_ref, ox_ref, oy_ref, oz_ref, dists_ref):
    b, n = xs_ref.shape
    xs = xs_ref[...]
    ys = ys_ref[...]
    zs = zs_ref[...]
    lane = jax.lax.broadcasted_iota(jnp.int32, (1, n), 1)
    lane_o = jax.lax.broadcasted_iota(jnp.int32, (1, _NPOINT), 1)
    dists_ref[...] = jnp.full((b, n), 1e10, jnp.float32)

    def step(i, far):
        onehot = (lane == far).astype(jnp.float32)
        cx = jnp.sum(xs * onehot, axis=1, keepdims=True)
        cy = jnp.sum(ys * onehot, axis=1, keepdims=True)
        cz = jnp.sum(zs * onehot, axis=1, keepdims=True)
        d = (xs - cx) ** 2 + (ys - cy) ** 2 + (zs - cz) ** 2
        dmin = jnp.minimum(dists_ref[...], d)
        dists_ref[...] = dmin
        m = jnp.max(dmin, axis=1, keepdims=True)
        nxt = jnp.min(jnp.where(dmin == m, lane, n), axis=1, keepdims=True)
        sel = lane_o == i
        ox_ref[...] = jnp.where(sel, cx, ox_ref[...])
        oy_ref[...] = jnp.where(sel, cy, oy_ref[...])
        oz_ref[...] = jnp.where(sel, cz, oz_ref[...])
        return nxt

    far0 = jnp.zeros((b, 1), jnp.int32)
    jax.lax.fori_loop(0, _NPOINT, step, far0)


def _run_fps(xyz):
    b, n, _ = xyz.shape
    xs = xyz[:, :, 0]
    ys = xyz[:, :, 1]
    zs = xyz[:, :, 2]
    out = pl.pallas_call(
        _fps_body,
        out_shape=[jax.ShapeDtypeStruct((b, _NPOINT), jnp.float32)] * 3,
        scratch_shapes=[pltpu.VMEM((b, n), jnp.float32)],
    )(xs, ys, zs)
    return out


def _cumsum_2d(m):
    r, c = m.shape
    v = m
    sh = 1
    while sh < c:
        v = v + jnp.pad(v[:, : c - sh], ((0, 0), (sh, 0)))
        sh *= 2
    rowtot = v[:, c - 1:c]
    incl = rowtot
    sh = 1
    while sh < r:
        incl = incl + jnp.pad(incl[: r - sh], ((sh, 0), (0, 0)))
        sh *= 2
    return v + (incl - rowtot)


def _group_body(xs_ref, ys_ref, zs_ref, qx_ref, qy_ref, qz_ref, feat_ref,
                out_ref, *, s_blk):
    n = feat_ref.shape[1]
    xs = xs_ref[0, 0]
    ys = ys_ref[0, 0]
    zs = zs_ref[0, 0]
    lane = jax.lax.broadcasted_iota(jnp.int32, (1, n), 1)
    e0 = (lane == 0).astype(jnp.float32)
    ones = jnp.ones((1, n), jnp.float32)
    feat = feat_ref[0]

    rows = []
    qs = []
    for k in range(s_blk):
        qx = qx_ref[0, k, 0]
        qy = qy_ref[0, k, 0]
        qz = qz_ref[0, k, 0]
        qs.append((qx, qy, qz))
        d = (xs - qx) ** 2 + (ys - qy) ** 2 + (zs - qz) ** 2
        for r, ns in zip(_RADII, _NSAMPLES):
            mask = (d <= r * r).astype(jnp.float32)
            c2 = _cumsum_2d(mask)
            cnt = c2[n // 128 - 1, 127]
            cm = jnp.where(mask > 0.0, c2, 0.0).reshape(1, n)
            kv1 = jax.lax.broadcasted_iota(
                jnp.int32, (ns, 1), 0).astype(jnp.float32) + 1.0
            kk = jnp.where(kv1 <= cnt, kv1, jnp.minimum(cnt, 1.0))
            has = (cnt > 0.0).astype(jnp.float32)
            w = has * ones + (1.0 - has) * e0
            a = (cm == kk).astype(jnp.float32) * w
            rows.append(a.astype(jnp.bfloat16))
    amat = jnp.concatenate(rows, axis=0)
    g3 = jax.lax.dot_general(
        amat, feat, (((1,), (0,)), ((), ())),
        preferred_element_type=jnp.float32)
    g = g3[:, :35] + g3[:, 35:70] + g3[:, 70:105]
    for k in range(s_blk):
        qx, qy, qz = qs[k]
        q = jnp.concatenate(
            [jnp.full((1, 1), qx, jnp.float32),
             jnp.full((1, 1), qy, jnp.float32),
             jnp.full((1, 1), qz, jnp.float32)], axis=1)
        out_ref[0, k] = g[k * _NS_TOT:(k + 1) * _NS_TOT] - jnp.pad(
            q, ((0, 0), (32, 0)))


def _run_group(xs, ys, zs, qx, qy, qz, feat, s_blk=8):
    b, n = xs.shape
    rr = n // 128
    grid = (b, _NPOINT // s_blk)
    kern = functools.partial(_group_body, s_blk=s_blk)
    out = pl.pallas_call(
        kern,
        grid=grid,
        in_specs=[
            pl.BlockSpec((1, 1, rr, 128), lambda i, j: (i, 0, 0, 0)),
            pl.BlockSpec((1, 1, rr, 128), lambda i, j: (i, 0, 0, 0)),
            pl.BlockSpec((1, 1, rr, 128), lambda i, j: (i, 0, 0, 0)),
            pl.BlockSpec((1, s_blk, 1), lambda i, j: (i, j, 0)),
            pl.BlockSpec((1, s_blk, 1), lambda i, j: (i, j, 0)),
            pl.BlockSpec((1, s_blk, 1), lambda i, j: (i, j, 0)),
            pl.BlockSpec((1, n, 105), lambda i, j: (i, 0, 0)),
        ],
        out_specs=pl.BlockSpec((1, s_blk, _NS_TOT, 35),
                               lambda i, j: (i, j, 0, 0)),
        out_shape=jax.ShapeDtypeStruct((b, _NPOINT, _NS_TOT, 35),
                                       jnp.float32),
    )(xs.reshape(b, 1, rr, 128), ys.reshape(b, 1, rr, 128),
      zs.reshape(b, 1, rr, 128),
      qx[:, :, None], qy[:, :, None], qz[:, :, None], feat)
    return out


def _conv_stats_body(x_ref, w_ref, y_ref, st_ref):
    i = pl.program_id(0)
    y = jax.lax.dot_general(
        x_ref[...], w_ref[...], (((1,), (0,)), ((), ())),
        preferred_element_type=jnp.float32)
    y_ref[...] = y
    s = jnp.sum(y, axis=0, keepdims=True)
    s2 = jnp.sum(y * y, axis=0, keepdims=True)
    st = jnp.concatenate([s, s2], axis=0)

    @pl.when(i == 0)
    def _():
        st_ref[...] = jnp.zeros_like(st_ref)

    st_ref[...] += st


def _norm_conv_stats_body(x_ref, st_in_ref, g_ref, b_ref, w_ref,
                          y_ref, st_ref, *, m_total):
    i = pl.program_id(0)
    st_in = st_in_ref[...]
    mean = st_in[0:1] / m_total
    var = st_in[1:2] / m_total - mean * mean
    scale = g_ref[...] * jax.lax.rsqrt(var + _EPS)
    bias = b_ref[...] - mean * scale
    z = jnp.maximum(x_ref[...] * scale + bias, 0.0)
    y = jax.lax.dot_general(
        z, w_ref[...], (((1,), (0,)), ((), ())),
        preferred_element_type=jnp.float32)
    y_ref[...] = y
    s = jnp.sum(y, axis=0, keepdims=True)
    s2 = jnp.sum(y * y, axis=0, keepdims=True)
    st = jnp.concatenate([s, s2], axis=0)

    @pl.when(i == 0)
    def _():
        st_ref[...] = jnp.zeros_like(st_ref)

    st_ref[...] += st


def _norm_pool_body(x_ref, st_in_ref, g_ref, b_ref, o_ref, *, m_total, ns):
    st_in = st_in_ref[...]
    mean = st_in[0:1] / m_total
    var = st_in[1:2] / m_total - mean * mean
    scale = g_ref[...] * jax.lax.rsqrt(var + _EPS)
    bias = b_ref[...] - mean * scale
    z = jnp.maximum(x_ref[...] * scale + bias, 0.0)
    rows, c = z.shape
    o_ref[...] = jnp.max(z.reshape(rows // ns, ns, c), axis=1)


def _conv_stats(x, w, blk):
    m, cin = x.shape
    cout = w.shape[1]
    grid = (m // blk,)
    return pl.pallas_call(
        _conv_stats_body,
        grid=grid,
        in_specs=[
            pl.BlockSpec((blk, cin), lambda i: (i, 0)),
            pl.BlockSpec((cin, cout), lambda i: (0, 0)),
        ],
        out_specs=[
            pl.BlockSpec((blk, cout), lambda i: (i, 0)),
            pl.BlockSpec((2, cout), lambda i: (0, 0)),
        ],
        out_shape=[
            jax.ShapeDtypeStruct((m, cout), jnp.float32),
            jax.ShapeDtypeStruct((2, cout), jnp.float32),
        ],
    )(x, w)


def _norm_conv_stats(x, st, gamma, beta, w, blk):
    m, cin = x.shape
    cout = w.shape[1]
    grid = (m // blk,)
    kern = functools.partial(_norm_conv_stats_body, m_total=float(m))
    return pl.pallas_call(
        kern,
        grid=grid,
        in_specs=[
            pl.BlockSpec((blk, cin), lambda i: (i, 0)),
            pl.BlockSpec((2, cin), lambda i: (0, 0)),
            pl.BlockSpec((1, cin), lambda i: (0, 0)),
            pl.BlockSpec((1, cin), lambda i: (0, 0)),
            pl.BlockSpec((cin, cout), lambda i: (0, 0)),
        ],
        out_specs=[
            pl.BlockSpec((blk, cout), lambda i: (i, 0)),
            pl.BlockSpec((2, cout), lambda i: (0, 0)),
        ],
        out_shape=[
            jax.ShapeDtypeStruct((m, cout), jnp.float32),
            jax.ShapeDtypeStruct((2, cout), jnp.float32),
        ],
    )(x, st, gamma, beta, w)


def _norm_pool(x, st, gamma, beta, ns, blk):
    m, c = x.shape
    grid = (m // blk,)
    kern = functools.partial(_norm_pool_body, m_total=float(m), ns=ns)
    return pl.pallas_call(
        kern,
        grid=grid,
        in_specs=[
            pl.BlockSpec((blk, c), lambda i: (i, 0)),
            pl.BlockSpec((2, c), lambda i: (0, 0)),
            pl.BlockSpec((1, c), lambda i: (0, 0)),
            pl.BlockSpec((1, c), lambda i: (0, 0)),
        ],
        out_specs=pl.BlockSpec((blk // ns, c), lambda i: (i, 0)),
        out_shape=jax.ShapeDtypeStruct((m // ns, c), jnp.float32),
    )(x, st, gamma, beta)


def kernel(xyz, points, params):
    b, n, _ = xyz.shape
    ox, oy, oz = _run_fps(xyz)

    feat = jnp.concatenate([points, xyz], axis=-1)
    e1 = feat.astype(jnp.bfloat16)
    r1 = feat - e1.astype(jnp.float32)
    e2 = r1.astype(jnp.bfloat16)
    e3 = (r1 - e2.astype(jnp.float32)).astype(jnp.bfloat16)
    featp = jnp.concatenate([e1, e2, e3], axis=-1)
    grouped = _run_group(xyz[:, :, 0], xyz[:, :, 1], xyz[:, :, 2],
                         ox, oy, oz, featp)

    outs = []
    off = 0
    for si, ns in enumerate(_NSAMPLES):
        x = grouped[:, :, off:off + ns, :].reshape(b * _NPOINT * ns, 35)
        off += ns
        blk = min(4096, x.shape[0])
        p0, p1, p2 = params[si]
        y, st = _conv_stats(x, p0["W"], blk)
        y, st2 = _norm_conv_stats(y, st, p0["gamma"][None], p0["beta"][None],
                                  p1["W"], blk)
        y, st3 = _norm_conv_stats(y, st2, p1["gamma"][None], p1["beta"][None],
                                  p2["W"], blk)
        o = _norm_pool(y, st3, p2["gamma"][None], p2["beta"][None], ns, blk)
        outs.append(o.reshape(b, _NPOINT, -1))

    new_xyz = jnp.stack([ox, oy, oz], axis=-1)
    return new_xyz, jnp.concatenate(outs, axis=-1)


# speedup vs baseline: 5.9184x; 1.1197x over previous
import functools

import jax
import jax.numpy as jnp
from jax.experimental import pallas as pl
from jax.experimental.pallas import tpu as pltpu

_NPOINT = 1024
_RADII = (0.1, 0.2, 0.4)
_NSAMPLES = (16, 32, 64)
_NS_TOT = 16 + 32 + 64
_EPS = 1e-3


def _fps_body(xs_ref, ys_ref, zs_ref, ox_ref, oy_ref, oz_ref, dists_ref):
    b, n = xs_ref.shape
    xs = xs_ref[...]
    ys = ys_ref[...]
    zs = zs_ref[...]
    lane = jax.lax.broadcasted_iota(jnp.int32, (1, n), 1)
    lane_o = jax.lax.broadcasted_iota(jnp.int32, (1, _NPOINT), 1)
    dists_ref[...] = jnp.full((b, n), 1e10, jnp.float32)

    def step(i, far):
        onehot = (lane == far).astype(jnp.float32)
        cx = jnp.sum(xs * onehot, axis=1, keepdims=True)
        cy = jnp.sum(ys * onehot, axis=1, keepdims=True)
        cz = jnp.sum(zs * onehot, axis=1, keepdims=True)
        d = (xs - cx) ** 2 + (ys - cy) ** 2 + (zs - cz) ** 2
        dmin = jnp.minimum(dists_ref[...], d)
        dists_ref[...] = dmin
        m = jnp.max(dmin, axis=1, keepdims=True)
        nxt = jnp.min(jnp.where(dmin == m, lane, n), axis=1, keepdims=True)
        sel = lane_o == i
        ox_ref[...] = jnp.where(sel, cx, ox_ref[...])
        oy_ref[...] = jnp.where(sel, cy, oy_ref[...])
        oz_ref[...] = jnp.where(sel, cz, oz_ref[...])
        return nxt

    far0 = jnp.zeros((b, 1), jnp.int32)
    jax.lax.fori_loop(0, _NPOINT, step, far0)


def _run_fps(xyz):
    b, n, _ = xyz.shape
    xs = xyz[:, :, 0]
    ys = xyz[:, :, 1]
    zs = xyz[:, :, 2]
    out = pl.pallas_call(
        _fps_body,
        out_shape=[jax.ShapeDtypeStruct((b, _NPOINT), jnp.float32)] * 3,
        scratch_shapes=[pltpu.VMEM((b, n), jnp.float32)],
    )(xs, ys, zs)
    return out


def _cumsum_2d(m):
    r, c = m.shape
    v = m
    sh = 1
    while sh < c:
        v = v + jnp.pad(v[:, : c - sh], ((0, 0), (sh, 0)))
        sh *= 2
    rowtot = v[:, c - 1:c]
    incl = rowtot
    sh = 1
    while sh < r:
        incl = incl + jnp.pad(incl[: r - sh], ((sh, 0), (0, 0)))
        sh *= 2
    return v + (incl - rowtot)


def _group_body(xs_ref, ys_ref, zs_ref, qx_ref, qy_ref, qz_ref, feat_ref,
                out_ref, *, s_blk):
    n = feat_ref.shape[1]
    xs = xs_ref[0, 0]
    ys = ys_ref[0, 0]
    zs = zs_ref[0, 0]
    lane = jax.lax.broadcasted_iota(jnp.int32, (1, n), 1)
    e0 = (lane == 0).astype(jnp.float32)
    ones = jnp.ones((1, n), jnp.float32)
    feat = feat_ref[0]

    for k in range(s_blk):
        qx = qx_ref[0, k, 0]
        qy = qy_ref[0, k, 0]
        qz = qz_ref[0, k, 0]
        d = (xs - qx) ** 2 + (ys - qy) ** 2 + (zs - qz) ** 2
        rows = []
        for r, ns in zip(_RADII, _NSAMPLES):
            mask = (d <= r * r).astype(jnp.float32)
            c2 = _cumsum_2d(mask)
            cnt = c2[n // 128 - 1, 127]
            cm = jnp.where(mask > 0.0, c2, 0.0)
            cmb = jnp.minimum(cm, 128.0).reshape(1, n).astype(jnp.bfloat16)
            kv1 = jax.lax.broadcasted_iota(
                jnp.int32, (ns, 1), 0).astype(jnp.float32) + 1.0
            kk = jnp.where(kv1 <= cnt, kv1, jnp.minimum(cnt, 1.0))
            has = (cnt > 0.0).astype(jnp.float32)
            w = (has * ones + (1.0 - has) * e0).astype(jnp.bfloat16)
            a = jnp.where(cmb == kk.astype(jnp.bfloat16), w,
                          jnp.bfloat16(0.0))
            rows.append(a)
        amat = jnp.concatenate(rows, axis=0)
        g3 = jax.lax.dot_general(
            amat, feat, (((1,), (0,)), ((), ())),
            preferred_element_type=jnp.float32)
        g = g3[:, :35] + g3[:, 35:70] + g3[:, 70:105]
        q = jnp.concatenate(
            [jnp.full((1, 1), qx, jnp.float32),
             jnp.full((1, 1), qy, jnp.float32),
             jnp.full((1, 1), qz, jnp.float32)], axis=1)
        out_ref[0, k] = g - jnp.pad(q, ((0, 0), (32, 0)))


def _run_group(xs, ys, zs, qx, qy, qz, feat, s_blk=8):
    b, n = xs.shape
    rr = n // 128
    grid = (b, _NPOINT // s_blk)
    kern = functools.partial(_group_body, s_blk=s_blk)
    out = pl.pallas_call(
        kern,
        grid=grid,
        in_specs=[
            pl.BlockSpec((1, 1, rr, 128), lambda i, j: (i, 0, 0, 0)),
            pl.BlockSpec((1, 1, rr, 128), lambda i, j: (i, 0, 0, 0)),
            pl.BlockSpec((1, 1, rr, 128), lambda i, j: (i, 0, 0, 0)),
            pl.BlockSpec((1, s_blk, 1), lambda i, j: (i, j, 0)),
            pl.BlockSpec((1, s_blk, 1), lambda i, j: (i, j, 0)),
            pl.BlockSpec((1, s_blk, 1), lambda i, j: (i, j, 0)),
            pl.BlockSpec((1, n, 105), lambda i, j: (i, 0, 0)),
        ],
        out_specs=pl.BlockSpec((1, s_blk, _NS_TOT, 35),
                               lambda i, j: (i, j, 0, 0)),
        out_shape=jax.ShapeDtypeStruct((b, _NPOINT, _NS_TOT, 35),
                                       jnp.float32),
    )(xs.reshape(b, 1, rr, 128), ys.reshape(b, 1, rr, 128),
      zs.reshape(b, 1, rr, 128),
      qx[:, :, None], qy[:, :, None], qz[:, :, None], feat)
    return out


def _conv_stats_body(x_ref, w_ref, y_ref, st_ref):
    i = pl.program_id(0)
    y = jax.lax.dot_general(
        x_ref[...], w_ref[...], (((1,), (0,)), ((), ())),
        preferred_element_type=jnp.float32)
    y_ref[...] = y
    s = jnp.sum(y, axis=0, keepdims=True)
    s2 = jnp.sum(y * y, axis=0, keepdims=True)
    st = jnp.concatenate([s, s2], axis=0)

    @pl.when(i == 0)
    def _():
        st_ref[...] = jnp.zeros_like(st_ref)

    st_ref[...] += st


def _norm_conv_stats_body(x_ref, st_in_ref, g_ref, b_ref, w_ref,
                          y_ref, st_ref, *, m_total):
    i = pl.program_id(0)
    st_in = st_in_ref[...]
    mean = st_in[0:1] / m_total
    var = st_in[1:2] / m_total - mean * mean
    scale = g_ref[...] * jax.lax.rsqrt(var + _EPS)
    bias = b_ref[...] - mean * scale
    z = jnp.maximum(x_ref[...] * scale + bias, 0.0)
    y = jax.lax.dot_general(
        z, w_ref[...], (((1,), (0,)), ((), ())),
        preferred_element_type=jnp.float32)
    y_ref[...] = y
    s = jnp.sum(y, axis=0, keepdims=True)
    s2 = jnp.sum(y * y, axis=0, keepdims=True)
    st = jnp.concatenate([s, s2], axis=0)

    @pl.when(i == 0)
    def _():
        st_ref[...] = jnp.zeros_like(st_ref)

    st_ref[...] += st


def _norm_pool_body(x_ref, st_in_ref, g_ref, b_ref, o_ref, *, m_total, ns):
    st_in = st_in_ref[...]
    mean = st_in[0:1] / m_total
    var = st_in[1:2] / m_total - mean * mean
    scale = g_ref[...] * jax.lax.rsqrt(var + _EPS)
    bias = b_ref[...] - mean * scale
    z = jnp.maximum(x_ref[...] * scale + bias, 0.0)
    rows, c = z.shape
    o_ref[...] = jnp.max(z.reshape(rows // ns, ns, c), axis=1)


def _conv_stats(x, w, blk):
    m, cin = x.shape
    cout = w.shape[1]
    grid = (m // blk,)
    return pl.pallas_call(
        _conv_stats_body,
        grid=grid,
        in_specs=[
            pl.BlockSpec((blk, cin), lambda i: (i, 0)),
            pl.BlockSpec((cin, cout), lambda i: (0, 0)),
        ],
        out_specs=[
            pl.BlockSpec((blk, cout), lambda i: (i, 0)),
            pl.BlockSpec((2, cout), lambda i: (0, 0)),
        ],
        out_shape=[
            jax.ShapeDtypeStruct((m, cout), jnp.float32),
            jax.ShapeDtypeStruct((2, cout), jnp.float32),
        ],
    )(x, w)


def _norm_conv_stats(x, st, gamma, beta, w, blk):
    m, cin = x.shape
    cout = w.shape[1]
    grid = (m // blk,)
    kern = functools.partial(_norm_conv_stats_body, m_total=float(m))
    return pl.pallas_call(
        kern,
        grid=grid,
        in_specs=[
            pl.BlockSpec((blk, cin), lambda i: (i, 0)),
            pl.BlockSpec((2, cin), lambda i: (0, 0)),
            pl.BlockSpec((1, cin), lambda i: (0, 0)),
            pl.BlockSpec((1, cin), lambda i: (0, 0)),
            pl.BlockSpec((cin, cout), lambda i: (0, 0)),
        ],
        out_specs=[
            pl.BlockSpec((blk, cout), lambda i: (i, 0)),
            pl.BlockSpec((2, cout), lambda i: (0, 0)),
        ],
        out_shape=[
            jax.ShapeDtypeStruct((m, cout), jnp.float32),
            jax.ShapeDtypeStruct((2, cout), jnp.float32),
        ],
    )(x, st, gamma, beta, w)


def _norm_pool(x, st, gamma, beta, ns, blk):
    m, c = x.shape
    grid = (m // blk,)
    kern = functools.partial(_norm_pool_body, m_total=float(m), ns=ns)
    return pl.pallas_call(
        kern,
        grid=grid,
        in_specs=[
            pl.BlockSpec((blk, c), lambda i: (i, 0)),
            pl.BlockSpec((2, c), lambda i: (0, 0)),
            pl.BlockSpec((1, c), lambda i: (0, 0)),
            pl.BlockSpec((1, c), lambda i: (0, 0)),
        ],
        out_specs=pl.BlockSpec((blk // ns, c), lambda i: (i, 0)),
        out_shape=jax.ShapeDtypeStruct((m // ns, c), jnp.float32),
    )(x, st, gamma, beta)


def kernel(xyz, points, params):
    b, n, _ = xyz.shape
    ox, oy, oz = _run_fps(xyz)

    feat = jnp.concatenate([points, xyz], axis=-1)
    e1 = feat.astype(jnp.bfloat16)
    r1 = feat - e1.astype(jnp.float32)
    e2 = r1.astype(jnp.bfloat16)
    e3 = (r1 - e2.astype(jnp.float32)).astype(jnp.bfloat16)
    featp = jnp.concatenate([e1, e2, e3], axis=-1)
    grouped = _run_group(xyz[:, :, 0], xyz[:, :, 1], xyz[:, :, 2],
                         ox, oy, oz, featp)

    outs = []
    off = 0
    for si, ns in enumerate(_NSAMPLES):
        x = grouped[:, :, off:off + ns, :].reshape(b * _NPOINT * ns, 35)
        off += ns
        blk = min(4096, x.shape[0])
        p0, p1, p2 = params[si]
        y, st = _conv_stats(x, p0["W"], blk)
        y, st2 = _norm_conv_stats(y, st, p0["gamma"][None], p0["beta"][None],
                                  p1["W"], blk)
        y, st3 = _norm_conv_stats(y, st2, p1["gamma"][None], p1["beta"][None],
                                  p2["W"], blk)
        o = _norm_pool(y, st3, p2["gamma"][None], p2["beta"][None], ns, blk)
        outs.append(o.reshape(b, _NPOINT, -1))

    new_xyz = jnp.stack([ox, oy, oz], axis=-1)
    return new_xyz, jnp.concatenate(outs, axis=-1)
